# Initial kernel scaffold; baseline (speedup 1.0000x reference)
import jax
import jax.numpy as jnp
from jax.experimental import pallas as pl


def kernel(ctrl_tokens, prev_trace, embed_table):
    raise NotImplementedError("write your pallas kernel here")


# speedup vs baseline: 23.2522x; 23.2522x over previous
import functools

import jax
import jax.numpy as jnp
from jax import lax
from jax.experimental import pallas as pl
from jax.experimental.pallas import tpu as pltpu
from jax.experimental.pallas import tpu_sc as plsc

BATCH = 1024
T_STEPS = 200
D_DIM = 128
DECAY = 0.9

NUM_CORES = 2
NUM_SUBCORES = 16
NUM_WORKERS = NUM_CORES * NUM_SUBCORES
ROWS_PER_WORKER = BATCH // NUM_WORKERS
LANES = 16
DC = D_DIM // LANES


def kernel(ctrl_tokens, prev_trace, embed_table):
    idx = ctrl_tokens[:, :, 1].astype(jnp.int32)

    mesh = plsc.VectorSubcoreMesh(core_axis_name="c", subcore_axis_name="s")

    @functools.partial(
        pl.kernel,
        out_type=jax.ShapeDtypeStruct((BATCH, T_STEPS, D_DIM), jnp.float32),
        mesh=mesh,
        scratch_types=[
            pltpu.VMEM((T_STEPS,), jnp.int32),
            pltpu.VMEM((T_STEPS, D_DIM), jnp.float32),
            pltpu.VMEM((ROWS_PER_WORKER, D_DIM), jnp.float32),
            pltpu.SemaphoreType.DMA,
        ],
    )
    def ev_kernel(idx_hbm, prev_hbm, table_hbm, out_hbm,
                  idx_v, rows_v, prev_v, sem):
        wid = lax.axis_index("s") * NUM_CORES + lax.axis_index("c")
        base = wid * ROWS_PER_WORKER
        pltpu.sync_copy(prev_hbm.at[pl.ds(base, ROWS_PER_WORKER)], prev_v)

        @pl.loop(0, ROWS_PER_WORKER)
        def _(r):
            row = base + r
            pltpu.sync_copy(idx_hbm.at[row], idx_v)
            pltpu.async_copy(table_hbm.at[idx_v], rows_v, sem).wait()

            def step(t, acc):
                new = tuple(
                    rows_v[t, pl.ds(c * LANES, LANES)] + DECAY * acc[c]
                    for c in range(DC)
                )
                for c in range(DC):
                    rows_v[t, pl.ds(c * LANES, LANES)] = new[c]
                return new

            acc0 = tuple(prev_v[r, pl.ds(c * LANES, LANES)] for c in range(DC))
            lax.fori_loop(0, T_STEPS, step, acc0)
            pltpu.sync_copy(rows_v, out_hbm.at[row])

    return ev_kernel(idx, prev_trace, embed_table)


# speedup vs baseline: 27.5524x; 1.1849x over previous
import functools

import jax
import jax.numpy as jnp
from jax import lax
from jax.experimental import pallas as pl
from jax.experimental.pallas import tpu as pltpu
from jax.experimental.pallas import tpu_sc as plsc

BATCH = 1024
T_STEPS = 200
D_DIM = 128
DECAY = 0.9

NUM_CORES = 2
NUM_SUBCORES = 16
NUM_WORKERS = NUM_CORES * NUM_SUBCORES
ROWS_PER_WORKER = BATCH // NUM_WORKERS
LANES = 16
DC = D_DIM // LANES
NBUF = 4


def kernel(ctrl_tokens, prev_trace, embed_table):
    idx = ctrl_tokens[:, :, 1].astype(jnp.int32).reshape(BATCH * T_STEPS)

    mesh = plsc.VectorSubcoreMesh(core_axis_name="c", subcore_axis_name="s")

    @functools.partial(
        pl.kernel,
        out_type=jax.ShapeDtypeStruct((BATCH, T_STEPS, D_DIM), jnp.float32),
        mesh=mesh,
        scratch_types=[
            pltpu.VMEM((ROWS_PER_WORKER * T_STEPS,), jnp.int32),
            pltpu.VMEM((NBUF, T_STEPS, D_DIM), jnp.float32),
            pltpu.VMEM((ROWS_PER_WORKER, D_DIM), jnp.float32),
            pltpu.SemaphoreType.DMA((NBUF,)),
            pltpu.SemaphoreType.DMA((NBUF,)),
        ],
    )
    def ev_kernel(idx_hbm, prev_hbm, table_hbm, out_hbm,
                  idx_v, rows_v, prev_v, gsem, osem):
        wid = lax.axis_index("s") * NUM_CORES + lax.axis_index("c")
        base = wid * ROWS_PER_WORKER
        pltpu.sync_copy(
            idx_hbm.at[pl.ds(base * T_STEPS, ROWS_PER_WORKER * T_STEPS)], idx_v)
        pltpu.sync_copy(prev_hbm.at[pl.ds(base, ROWS_PER_WORKER)], prev_v)

        def gather(r, b):
            return pltpu.make_async_copy(
                table_hbm.at[idx_v.at[pl.ds(r * T_STEPS, T_STEPS)]],
                rows_v.at[b], gsem.at[b])

        def out_copy(r, b):
            return pltpu.make_async_copy(
                rows_v.at[b], out_hbm.at[base + r], osem.at[b])

        gather(0, 0).start()
        gather(1, 1).start()

        @pl.loop(0, ROWS_PER_WORKER, step=NBUF)
        def _(rbase):
            for j in range(NBUF):
                b = j
                pb = (j + 2) % NBUF
                r = rbase + j

                @pl.when(r < ROWS_PER_WORKER - 2)
                def _():
                    @pl.when(r >= 2)
                    def _():
                        out_copy(r - 2, pb).wait()

                    gather(r + 2, pb).start()

                gather(r, b).wait()

                def step(t, acc):
                    new = tuple(
                        rows_v[b, t, pl.ds(c * LANES, LANES)] + DECAY * acc[c]
                        for c in range(DC)
                    )
                    for c in range(DC):
                        rows_v[b, t, pl.ds(c * LANES, LANES)] = new[c]
                    return new

                acc0 = tuple(
                    prev_v[r, pl.ds(c * LANES, LANES)] for c in range(DC))
                lax.fori_loop(0, T_STEPS, step, acc0)

                out_copy(r, b).start()

        for b in range(NBUF):
            out_copy(ROWS_PER_WORKER - NBUF + b, b).wait()

    return ev_kernel(idx, prev_trace, embed_table)


# speedup vs baseline: 27.7471x; 1.0071x over previous
import functools

import jax
import jax.numpy as jnp
from jax import lax
from jax.experimental import pallas as pl
from jax.experimental.pallas import tpu as pltpu
from jax.experimental.pallas import tpu_sc as plsc

BATCH = 1024
T_STEPS = 200
D_DIM = 128
DECAY = 0.9

NUM_CORES = 2
NUM_SUBCORES = 16
NUM_WORKERS = NUM_CORES * NUM_SUBCORES
ROWS_PER_WORKER = BATCH // NUM_WORKERS
LANES = 16
DC = D_DIM // LANES
NBUF = 4


def kernel(ctrl_tokens, prev_trace, embed_table):
    idx = ctrl_tokens[:, :, 1].astype(jnp.int32).reshape(BATCH * T_STEPS)

    mesh = plsc.VectorSubcoreMesh(core_axis_name="c", subcore_axis_name="s")

    @functools.partial(
        pl.kernel,
        out_type=jax.ShapeDtypeStruct((BATCH, T_STEPS, D_DIM), jnp.float32),
        mesh=mesh,
        scratch_types=[
            pltpu.VMEM((ROWS_PER_WORKER * T_STEPS,), jnp.int32),
            pltpu.VMEM((NBUF, T_STEPS, D_DIM), jnp.float32),
            pltpu.VMEM((ROWS_PER_WORKER, D_DIM), jnp.float32),
            pltpu.SemaphoreType.DMA((NBUF,)),
            pltpu.SemaphoreType.DMA((NBUF,)),
        ],
    )
    def ev_kernel(idx_hbm, prev_hbm, table_hbm, out_hbm,
                  idx_v, rows_v, prev_v, gsem, osem):
        wid = lax.axis_index("s") * NUM_CORES + lax.axis_index("c")
        base = wid * ROWS_PER_WORKER
        pltpu.sync_copy(
            idx_hbm.at[pl.ds(base * T_STEPS, ROWS_PER_WORKER * T_STEPS)], idx_v)
        pltpu.sync_copy(prev_hbm.at[pl.ds(base, ROWS_PER_WORKER)], prev_v)

        def gather(r, b):
            return pltpu.make_async_copy(
                table_hbm.at[idx_v.at[pl.ds(r * T_STEPS, T_STEPS)]],
                rows_v.at[b], gsem.at[b])

        def out_copy(r, b):
            return pltpu.make_async_copy(
                rows_v.at[b], out_hbm.at[base + r], osem.at[b])

        gather(0, 0).start()
        gather(1, 1).start()

        @pl.loop(0, ROWS_PER_WORKER, step=NBUF)
        def _(rbase):
            for j in range(NBUF):
                b = j
                pb = (j + 2) % NBUF
                r = rbase + j

                @pl.when(r < ROWS_PER_WORKER - 2)
                def _():
                    @pl.when(r >= 2)
                    def _():
                        out_copy(r - 2, pb).wait()

                    gather(r + 2, pb).start()

                gather(r, b).wait()

                out_copy(r, b).start()

        for b in range(NBUF):
            out_copy(ROWS_PER_WORKER - NBUF + b, b).wait()

    return ev_kernel(idx, prev_trace, embed_table)


# speedup vs baseline: 51.1202x; 1.8424x over previous
import functools

import jax
import jax.numpy as jnp
from jax import lax
from jax.experimental import pallas as pl
from jax.experimental.pallas import tpu as pltpu
from jax.experimental.pallas import tpu_sc as plsc

BATCH = 1024
VOCAB = 1000
T_STEPS = 200
D_DIM = 128
DECAY = 0.9

NUM_CORES = 2
NUM_SUBCORES = 16
NUM_WORKERS = NUM_CORES * NUM_SUBCORES
ROWS_PER_WORKER = BATCH // NUM_WORKERS
LANES = 16
DC = D_DIM // LANES
NBUF = 4


def kernel(ctrl_tokens, prev_trace, embed_table):
    idx = ctrl_tokens[:, :, 1].astype(jnp.int32).reshape(BATCH * T_STEPS)

    mesh = plsc.VectorSubcoreMesh(core_axis_name="c", subcore_axis_name="s")

    @functools.partial(
        pl.kernel,
        out_type=jax.ShapeDtypeStruct((BATCH, T_STEPS, D_DIM), jnp.float32),
        mesh=mesh,
        scratch_types=[
            pltpu.VMEM((ROWS_PER_WORKER * T_STEPS,), jnp.int32),
            pltpu.VMEM((NBUF, T_STEPS, D_DIM), jnp.float32),
            pltpu.VMEM((ROWS_PER_WORKER, D_DIM), jnp.float32),
            pltpu.SemaphoreType.DMA((NBUF,)),
            pltpu.SemaphoreType.DMA((NBUF,)),
            pltpu.VMEM_SHARED((VOCAB, D_DIM), jnp.float32),
        ],
    )
    def ev_kernel(idx_hbm, prev_hbm, table_hbm, out_hbm,
                  idx_v, rows_v, prev_v, gsem, osem, table_sh):
        wid = lax.axis_index("s") * NUM_CORES + lax.axis_index("c")
        base = wid * ROWS_PER_WORKER
        @pl.when(lax.axis_index("s") == 0)
        def _():
            pltpu.sync_copy(table_hbm, table_sh)
        plsc.subcore_barrier()
        pltpu.sync_copy(
            idx_hbm.at[pl.ds(base * T_STEPS, ROWS_PER_WORKER * T_STEPS)], idx_v)
        pltpu.sync_copy(prev_hbm.at[pl.ds(base, ROWS_PER_WORKER)], prev_v)

        def gather(r, b):
            return pltpu.make_async_copy(
                table_sh.at[idx_v.at[pl.ds(r * T_STEPS, T_STEPS)]],
                rows_v.at[b], gsem.at[b])

        def out_copy(r, b):
            return pltpu.make_async_copy(
                rows_v.at[b], out_hbm.at[base + r], osem.at[b])

        gather(0, 0).start()
        gather(1, 1).start()

        @pl.loop(0, ROWS_PER_WORKER, step=NBUF)
        def _(rbase):
            for j in range(NBUF):
                b = j
                pb = (j + 2) % NBUF
                r = rbase + j

                @pl.when(r < ROWS_PER_WORKER - 2)
                def _():
                    @pl.when(r >= 2)
                    def _():
                        out_copy(r - 2, pb).wait()

                    gather(r + 2, pb).start()

                gather(r, b).wait()

                def step(t, acc):
                    new = tuple(
                        rows_v[b, t, pl.ds(c * LANES, LANES)] + DECAY * acc[c]
                        for c in range(DC)
                    )
                    for c in range(DC):
                        rows_v[b, t, pl.ds(c * LANES, LANES)] = new[c]
                    return new

                acc0 = tuple(
                    prev_v[r, pl.ds(c * LANES, LANES)] for c in range(DC))
                lax.fori_loop(0, T_STEPS, step, acc0)

                out_copy(r, b).start()

        for b in range(NBUF):
            out_copy(ROWS_PER_WORKER - NBUF + b, b).wait()

    return ev_kernel(idx, prev_trace, embed_table)
